# baseline (speedup 1.0000x reference)
import functools

import jax
import jax.numpy as jnp
from jax import lax
from jax.experimental import pallas as pl
from jax.experimental.pallas import tpu as pltpu
from jax.experimental.pallas import tpu_sc as plsc

NC = 2
NS = 16
NW = NC * NS

CHUNK = 512
GATHER = 128
PADW = 128

GCOLS = 512
NG = 61


def _transpose_group(in_v, out_v, ncol):
    fe_lo = lax.iota(jnp.int32, 16)
    fe_hi = fe_lo + 16

    @plsc.parallel_loop(0, ncol // 4, unroll=4)
    def _(cb):
        for dc in range(4):
            cv = jnp.zeros((16,), jnp.int32) + (4 * cb + dc)
            v0 = plsc.load_gather(in_v, [fe_lo, cv])
            v1 = plsc.load_gather(in_v, [fe_hi, cv])
            out_v[cb, pl.ds(32 * dc, 16)] = v0
            out_v[cb, pl.ds(32 * dc + 16, 16)] = v1


def _retile_body(V, tT_hbm, trm_hbm, in0, in1, out0, out1, inp, outp,
                 si0, si1, so0, so1, sp):
    wid = lax.axis_index("s") * NC + lax.axis_index("c")
    vfull = (V // GCOLS) * GCOLS

    def gcol(t):
        return (wid + 32 * t) * GCOLS

    def load(t, iv, si):
        pltpu.async_copy(tT_hbm.at[:, pl.ds(gcol(t), GCOLS)], iv, si)

    def wait_load(iv, si):
        pltpu.make_async_copy(tT_hbm.at[:, pl.ds(0, GCOLS)], iv, si).wait()

    def wait_write(ov, so):
        pltpu.make_async_copy(trm_hbm.at[pl.ds(0, GCOLS // 4), :], ov,
                              so).wait()

    def process(t, iv, ov, si, so):
        wait_load(iv, si)
        _transpose_group(iv, ov, GCOLS)
        pltpu.async_copy(
            ov, trm_hbm.at[pl.ds((gcol(t) // GCOLS) * (GCOLS // 4),
                                 GCOLS // 4), :], so)

    load(0, in0, si0)

    def loop_body(i, carry):
        t0 = 2 * i
        load(t0 + 1, in1, si1)

        @pl.when(i > 0)
        def _():
            wait_write(out0, so0)
        process(t0, in0, out0, si0, so0)

        @pl.when(t0 + 2 < NG)
        def _():
            load(t0 + 2, in0, si0)

        @pl.when(i > 0)
        def _():
            wait_write(out1, so1)
        process(t0 + 1, in1, out1, si1, so1)
        return carry

    lax.fori_loop(0, NG // 2, loop_body, jnp.int32(0))
    wait_write(out0, so0)
    process(NG - 1, in0, out0, si0, so0)
    wait_write(out0, so0)
    wait_write(out1, so1)

    @pl.when(wid == 0)
    def _():
        pltpu.sync_copy(tT_hbm.at[:, pl.ds(vfull - GCOLS, GCOLS)], in0)
        _transpose_group(in0, out0, GCOLS)
        pltpu.sync_copy(
            out0, trm_hbm.at[pl.ds((vfull - GCOLS) // 4, GCOLS // 4), :])

        tail = V - vfull
        pltpu.sync_copy(tT_hbm.at[:, pl.ds(vfull, tail)], inp)
        _transpose_group(inp, outp, tail)
        pltpu.sync_copy(outp, trm_hbm.at[pl.ds(vfull // 4, tail // 4), :])


def _count_zeros(idx_v):
    ones = jnp.ones((16,), jnp.int32)
    zer = jnp.zeros((16,), jnp.int32)

    def cnt_body(i, acc):
        v = idx_v[pl.ds(i * 16, 16)]
        return acc + jnp.sum(jnp.where(v == 0, ones, zer))

    return lax.fori_loop(0, CHUNK // 16, cnt_body, jnp.int32(0))


def _fix_zero_rows(E, idx_v, rows_v):
    zeros = jnp.zeros((16,), jnp.float32)

    def fix_body(i, carry):
        v = idx_v[pl.ds(i * 16, 16)]
        m = v == 0
        rowids = lax.iota(jnp.int32, 16) + i * 16
        for col in range(E):
            colids = jnp.full((16,), col, jnp.int32)
            plsc.store_scatter(rows_v, [rowids, colids], zeros, mask=m)
        return carry

    lax.fori_loop(0, CHUNK // 16, fix_body, jnp.int32(0))


def _gather_body(E, cpl, nch, idx_hbm, table_hbm, out_hbm,
                 idx0, idx1, rows0, rows1, sg0, sg1, sw0, sw1):
    wid = lax.axis_index("s") * NC + lax.axis_index("c")
    cbase = wid * nch

    def load_and_fire(c, ib, rb, sg):
        pltpu.sync_copy(idx_hbm.at[pl.ds((cbase + c) * CHUNK, CHUNK)], ib)
        for j in range(CHUNK // GATHER):
            pltpu.async_copy(
                table_hbm.at[ib.at[pl.ds(j * GATHER, GATHER)]],
                rb.at[pl.ds(j * GATHER, GATHER)], sg)

    def wait_gathers(rb, sg):
        pltpu.make_async_copy(table_hbm.at[pl.ds(0, CHUNK)], rb, sg).wait()

    def wait_write(rb, sw):
        pltpu.make_async_copy(table_hbm.at[pl.ds(0, CHUNK)], rb, sw).wait()

    def process(c, nz, ib, rb, sg, sw):
        wait_gathers(rb, sg)

        @pl.when(nz > 0)
        def _():
            _fix_zero_rows(E, ib, rb)

        cg = cbase + c
        l = cg // cpl
        b0 = (cg % cpl) * CHUNK
        pltpu.async_copy(rb, out_hbm.at[l, pl.ds(b0, CHUNK), pl.ds(0, E)],
                         sw)

    def prefetch(c, first, last, ib, rb, sg, sw):
        @pl.when(jnp.logical_not(jnp.logical_or(first, last)))
        def _():
            wait_write(rb, sw)

        @pl.when(jnp.logical_not(last))
        def _():
            load_and_fire(c, ib, rb, sg)
        return _count_zeros(ib)

    nz0 = prefetch(0, jnp.bool_(True), jnp.bool_(False), idx0, rows0,
                   sg0, sw0)

    def loop_body(i, carry):
        nz0, nz1 = carry
        c0 = 2 * i
        nz1 = prefetch(c0 + 1, i == 0, jnp.bool_(False), idx1, rows1,
                       sg1, sw1)
        process(c0, nz0, idx0, rows0, sg0, sw0)
        nz0 = prefetch(c0 + 2, jnp.bool_(False), i == nch // 2 - 1,
                       idx0, rows0, sg0, sw0)
        process(c0 + 1, nz1, idx1, rows1, sg1, sw1)
        return nz0, nz1

    lax.fori_loop(0, nch // 2, loop_body, (nz0, nz0))

    wait_write(rows0, sw0)
    wait_write(rows1, sw1)


def kernel(input, table, max_steps):
    B, L = input.shape
    V, E = table.shape
    N = B * L
    cpl = B // CHUNK
    nch = N // CHUNK // NW

    idx_flat = input.T.reshape(N).astype(jnp.int32)

    mesh = plsc.VectorSubcoreMesh(core_axis_name="c", subcore_axis_name="s")

    trm = pl.kernel(
        functools.partial(_retile_body, V),
        out_type=jax.ShapeDtypeStruct((V // 4, 128), jnp.float32),
        mesh=mesh,
        compiler_params=pltpu.CompilerParams(use_tc_tiling_on_sc=True,
                                             needs_layout_passes=False),
        scratch_types=[
            pltpu.VMEM((32, GCOLS), jnp.float32),
            pltpu.VMEM((32, GCOLS), jnp.float32),
            pltpu.VMEM((GCOLS // 4, 128), jnp.float32),
            pltpu.VMEM((GCOLS // 4, 128), jnp.float32),
            pltpu.VMEM((32, 64), jnp.float32),
            pltpu.VMEM((16, 128), jnp.float32),
            pltpu.SemaphoreType.DMA,
            pltpu.SemaphoreType.DMA,
            pltpu.SemaphoreType.DMA,
            pltpu.SemaphoreType.DMA,
            pltpu.SemaphoreType.DMA,
        ],
    )(table.T)
    table_rm = trm.reshape(V, E)

    body = functools.partial(_gather_body, E, cpl, nch)
    out = pl.kernel(
        body,
        out_type=jax.ShapeDtypeStruct((L, B, PADW), jnp.float32),
        mesh=mesh,
        compiler_params=pltpu.CompilerParams(use_tc_tiling_on_sc=False,
                                             needs_layout_passes=False),
        scratch_types=[
            pltpu.VMEM((CHUNK,), jnp.int32),
            pltpu.VMEM((CHUNK,), jnp.int32),
            pltpu.VMEM((CHUNK, E), jnp.float32),
            pltpu.VMEM((CHUNK, E), jnp.float32),
            pltpu.SemaphoreType.DMA,
            pltpu.SemaphoreType.DMA,
            pltpu.SemaphoreType.DMA,
            pltpu.SemaphoreType.DMA,
        ],
    )(idx_flat, table_rm)
    return out[:, :, :E]


# speedup vs baseline: 1.0406x; 1.0406x over previous
import functools

import jax
import jax.numpy as jnp
from jax import lax
from jax.experimental import pallas as pl
from jax.experimental.pallas import tpu as pltpu
from jax.experimental.pallas import tpu_sc as plsc

NC = 2
NS = 16
NW = NC * NS

CHUNK = 512
GATHER = 128
PADW = 128

GCOLS = 512
NG = 61


def _transpose_group(in_v, out_v, ncol):
    lanes = lax.iota(jnp.int32, 16)
    addr_k = [512 * k + 32 * lanes for k in range(ncol // 16)]

    @plsc.parallel_loop(0, 32, unroll=2)
    def _(fe):
        for k in range(ncol // 16):
            v = in_v[fe, pl.ds(16 * k, 16)]
            plsc.store_scatter(out_v, [addr_k[k] + fe], v)


def _retile_body(V, tT_hbm, trm_hbm, in0, in1, out0, out1, inp, outp,
                 si0, si1, so0, so1, sp):
    wid = lax.axis_index("s") * NC + lax.axis_index("c")
    vfull = (V // GCOLS) * GCOLS

    def gcol(t):
        return (wid + 32 * t) * GCOLS

    def load(t, iv, si):
        pltpu.async_copy(tT_hbm.at[:, pl.ds(gcol(t), GCOLS)], iv, si)

    def wait_load(iv, si):
        pltpu.make_async_copy(tT_hbm.at[:, pl.ds(0, GCOLS)], iv, si).wait()

    def wait_write(ov, so):
        pltpu.make_async_copy(trm_hbm.at[pl.ds(0, 32 * GCOLS)], ov,
                              so).wait()

    def process(t, iv, ov, si, so):
        wait_load(iv, si)
        _transpose_group(iv, ov, GCOLS)
        pltpu.async_copy(ov, trm_hbm.at[pl.ds(gcol(t) * 32, 32 * GCOLS)],
                         so)

    load(0, in0, si0)

    def loop_body(i, carry):
        t0 = 2 * i
        load(t0 + 1, in1, si1)

        @pl.when(i > 0)
        def _():
            wait_write(out0, so0)
        process(t0, in0, out0, si0, so0)

        @pl.when(t0 + 2 < NG)
        def _():
            load(t0 + 2, in0, si0)

        @pl.when(i > 0)
        def _():
            wait_write(out1, so1)
        process(t0 + 1, in1, out1, si1, so1)
        return carry

    lax.fori_loop(0, NG // 2, loop_body, jnp.int32(0))
    wait_write(out0, so0)
    process(NG - 1, in0, out0, si0, so0)
    wait_write(out0, so0)
    wait_write(out1, so1)

    @pl.when(wid == 0)
    def _():
        pltpu.sync_copy(tT_hbm.at[:, pl.ds(vfull - GCOLS, GCOLS)], in0)
        _transpose_group(in0, out0, GCOLS)
        pltpu.sync_copy(out0,
                        trm_hbm.at[pl.ds((vfull - GCOLS) * 32, 32 * GCOLS)])

        tail = V - vfull
        pltpu.sync_copy(tT_hbm.at[:, pl.ds(vfull, tail)], inp)
        _transpose_group(inp, outp, tail)
        pltpu.sync_copy(outp, trm_hbm.at[pl.ds(vfull * 32, 32 * tail)])


def _count_zeros(idx_v):
    ones = jnp.ones((16,), jnp.int32)
    zer = jnp.zeros((16,), jnp.int32)

    def cnt_body(i, acc):
        v = idx_v[pl.ds(i * 16, 16)]
        return acc + jnp.sum(jnp.where(v == 0, ones, zer))

    return lax.fori_loop(0, CHUNK // 16, cnt_body, jnp.int32(0))


def _fix_zero_rows(E, idx_v, rows_v):
    zeros = jnp.zeros((16,), jnp.float32)

    def fix_body(i, carry):
        v = idx_v[pl.ds(i * 16, 16)]
        m = v == 0
        rowids = lax.iota(jnp.int32, 16) + i * 16
        for col in range(E):
            colids = jnp.full((16,), col, jnp.int32)
            plsc.store_scatter(rows_v, [rowids, colids], zeros, mask=m)
        return carry

    lax.fori_loop(0, CHUNK // 16, fix_body, jnp.int32(0))


def _gather_body(E, cpl, nch, idx_hbm, table_hbm, out_hbm,
                 idx0, idx1, rows0, rows1, sg0, sg1, sw0, sw1):
    wid = lax.axis_index("s") * NC + lax.axis_index("c")
    cbase = wid * nch

    def load_and_fire(c, ib, rb, sg):
        pltpu.sync_copy(idx_hbm.at[pl.ds((cbase + c) * CHUNK, CHUNK)], ib)
        for j in range(CHUNK // GATHER):
            pltpu.async_copy(
                table_hbm.at[ib.at[pl.ds(j * GATHER, GATHER)]],
                rb.at[pl.ds(j * GATHER, GATHER)], sg)

    def wait_gathers(rb, sg):
        pltpu.make_async_copy(table_hbm.at[pl.ds(0, CHUNK)], rb, sg).wait()

    def wait_write(rb, sw):
        pltpu.make_async_copy(table_hbm.at[pl.ds(0, CHUNK)], rb, sw).wait()

    def process(c, nz, ib, rb, sg, sw):
        wait_gathers(rb, sg)

        @pl.when(nz > 0)
        def _():
            _fix_zero_rows(E, ib, rb)

        cg = cbase + c
        l = cg // cpl
        b0 = (cg % cpl) * CHUNK
        pltpu.async_copy(rb, out_hbm.at[l, pl.ds(b0, CHUNK), pl.ds(0, E)],
                         sw)

    def prefetch(c, first, last, ib, rb, sg, sw):
        @pl.when(jnp.logical_not(jnp.logical_or(first, last)))
        def _():
            wait_write(rb, sw)

        @pl.when(jnp.logical_not(last))
        def _():
            load_and_fire(c, ib, rb, sg)
        return _count_zeros(ib)

    nz0 = prefetch(0, jnp.bool_(True), jnp.bool_(False), idx0, rows0,
                   sg0, sw0)

    def loop_body(i, carry):
        nz0, nz1 = carry
        c0 = 2 * i
        nz1 = prefetch(c0 + 1, i == 0, jnp.bool_(False), idx1, rows1,
                       sg1, sw1)
        process(c0, nz0, idx0, rows0, sg0, sw0)
        nz0 = prefetch(c0 + 2, jnp.bool_(False), i == nch // 2 - 1,
                       idx0, rows0, sg0, sw0)
        process(c0 + 1, nz1, idx1, rows1, sg1, sw1)
        return nz0, nz1

    lax.fori_loop(0, nch // 2, loop_body, (nz0, nz0))

    wait_write(rows0, sw0)
    wait_write(rows1, sw1)


def kernel(input, table, max_steps):
    B, L = input.shape
    V, E = table.shape
    N = B * L
    cpl = B // CHUNK
    nch = N // CHUNK // NW

    idx_flat = input.T.reshape(N).astype(jnp.int32)

    mesh = plsc.VectorSubcoreMesh(core_axis_name="c", subcore_axis_name="s")

    trm = pl.kernel(
        functools.partial(_retile_body, V),
        out_type=jax.ShapeDtypeStruct((V * E,), jnp.float32),
        mesh=mesh,
        compiler_params=pltpu.CompilerParams(use_tc_tiling_on_sc=True,
                                             needs_layout_passes=False),
        scratch_types=[
            pltpu.VMEM((32, GCOLS), jnp.float32),
            pltpu.VMEM((32, GCOLS), jnp.float32),
            pltpu.VMEM((32 * GCOLS,), jnp.float32),
            pltpu.VMEM((32 * GCOLS,), jnp.float32),
            pltpu.VMEM((32, 64), jnp.float32),
            pltpu.VMEM((32 * 64,), jnp.float32),
            pltpu.SemaphoreType.DMA,
            pltpu.SemaphoreType.DMA,
            pltpu.SemaphoreType.DMA,
            pltpu.SemaphoreType.DMA,
            pltpu.SemaphoreType.DMA,
        ],
    )(table.T)
    table_rm = trm.reshape(V, E)

    body = functools.partial(_gather_body, E, cpl, nch)
    out = pl.kernel(
        body,
        out_type=jax.ShapeDtypeStruct((L, B, PADW), jnp.float32),
        mesh=mesh,
        compiler_params=pltpu.CompilerParams(use_tc_tiling_on_sc=False,
                                             needs_layout_passes=False),
        scratch_types=[
            pltpu.VMEM((CHUNK,), jnp.int32),
            pltpu.VMEM((CHUNK,), jnp.int32),
            pltpu.VMEM((CHUNK, E), jnp.float32),
            pltpu.VMEM((CHUNK, E), jnp.float32),
            pltpu.SemaphoreType.DMA,
            pltpu.SemaphoreType.DMA,
            pltpu.SemaphoreType.DMA,
            pltpu.SemaphoreType.DMA,
        ],
    )(idx_flat, table_rm)
    return out[:, :, :E]


# speedup vs baseline: 1.1000x; 1.0571x over previous
import functools

import jax
import jax.numpy as jnp
from jax import lax
from jax.experimental import pallas as pl
from jax.experimental.pallas import tpu as pltpu
from jax.experimental.pallas import tpu_sc as plsc

NC = 2
NS = 16
NW = NC * NS

CHUNK = 512
GATHER = 128
PADW = 128

GCOLS = 512
NG = 61


def _transpose_group(in_v, out_v, ncol):
    lanes = lax.iota(jnp.int32, 16)
    fe_pat = lanes >> 2
    c_pat = lanes & 3
    dst_pat = c_pat * 32 + fe_pat

    def body(i, carry):
        f0 = 4 * i
        fe_v = fe_pat + f0
        for cb in range(ncol // 4):
            c0 = 4 * cb
            v = plsc.load_gather(in_v, [fe_v, c_pat + c0])
            plsc.store_scatter(out_v, [dst_pat + (c0 * 32 + f0)], v)
        return carry

    lax.fori_loop(0, 8, body, jnp.int32(0))


def _retile_body(V, tT_hbm, trm_hbm, in0, in1, out0, out1, inp, outp,
                 si0, si1, so0, so1, sp):
    wid = lax.axis_index("s") * NC + lax.axis_index("c")
    vfull = (V // GCOLS) * GCOLS

    def gcol(t):
        return (wid + 32 * t) * GCOLS

    def load(t, iv, si):
        pltpu.async_copy(tT_hbm.at[:, pl.ds(gcol(t), GCOLS)], iv, si)

    def wait_load(iv, si):
        pltpu.make_async_copy(tT_hbm.at[:, pl.ds(0, GCOLS)], iv, si).wait()

    def wait_write(ov, so):
        pltpu.make_async_copy(trm_hbm.at[pl.ds(0, 32 * GCOLS)], ov,
                              so).wait()

    def process(t, iv, ov, si, so):
        wait_load(iv, si)
        _transpose_group(iv, ov, GCOLS)
        pltpu.async_copy(ov, trm_hbm.at[pl.ds(gcol(t) * 32, 32 * GCOLS)],
                         so)

    load(0, in0, si0)

    def loop_body(i, carry):
        t0 = 2 * i
        load(t0 + 1, in1, si1)

        @pl.when(i > 0)
        def _():
            wait_write(out0, so0)
        process(t0, in0, out0, si0, so0)

        @pl.when(t0 + 2 < NG)
        def _():
            load(t0 + 2, in0, si0)

        @pl.when(i > 0)
        def _():
            wait_write(out1, so1)
        process(t0 + 1, in1, out1, si1, so1)
        return carry

    lax.fori_loop(0, NG // 2, loop_body, jnp.int32(0))
    wait_write(out0, so0)
    process(NG - 1, in0, out0, si0, so0)
    wait_write(out0, so0)
    wait_write(out1, so1)

    @pl.when(wid == 0)
    def _():
        pltpu.sync_copy(tT_hbm.at[:, pl.ds(vfull - GCOLS, GCOLS)], in0)
        _transpose_group(in0, out0, GCOLS)
        pltpu.sync_copy(out0,
                        trm_hbm.at[pl.ds((vfull - GCOLS) * 32, 32 * GCOLS)])

        tail = V - vfull
        pltpu.sync_copy(tT_hbm.at[:, pl.ds(vfull, tail)], inp)
        _transpose_group(inp, outp, tail)
        pltpu.sync_copy(outp, trm_hbm.at[pl.ds(vfull * 32, 32 * tail)])


def _count_zeros(idx_v):
    ones = jnp.ones((16,), jnp.int32)
    zer = jnp.zeros((16,), jnp.int32)

    def cnt_body(i, acc):
        v = idx_v[pl.ds(i * 16, 16)]
        return acc + jnp.sum(jnp.where(v == 0, ones, zer))

    return lax.fori_loop(0, CHUNK // 16, cnt_body, jnp.int32(0))


def _fix_zero_rows(E, idx_v, rows_v):
    zeros = jnp.zeros((16,), jnp.float32)

    def fix_body(i, carry):
        v = idx_v[pl.ds(i * 16, 16)]
        m = v == 0
        rowids = lax.iota(jnp.int32, 16) + i * 16
        for col in range(E):
            colids = jnp.full((16,), col, jnp.int32)
            plsc.store_scatter(rows_v, [rowids, colids], zeros, mask=m)
        return carry

    lax.fori_loop(0, CHUNK // 16, fix_body, jnp.int32(0))


def _gather_body(E, cpl, nch, idx_hbm, table_hbm, out_hbm,
                 idx0, idx1, rows0, rows1, sg0, sg1, sw0, sw1):
    wid = lax.axis_index("s") * NC + lax.axis_index("c")
    cbase = wid * nch

    def load_and_fire(c, ib, rb, sg):
        pltpu.sync_copy(idx_hbm.at[pl.ds((cbase + c) * CHUNK, CHUNK)], ib)
        for j in range(CHUNK // GATHER):
            pltpu.async_copy(
                table_hbm.at[ib.at[pl.ds(j * GATHER, GATHER)]],
                rb.at[pl.ds(j * GATHER, GATHER)], sg)

    def wait_gathers(rb, sg):
        pltpu.make_async_copy(table_hbm.at[pl.ds(0, CHUNK)], rb, sg).wait()

    def wait_write(rb, sw):
        pltpu.make_async_copy(table_hbm.at[pl.ds(0, CHUNK)], rb, sw).wait()

    def process(c, nz, ib, rb, sg, sw):
        wait_gathers(rb, sg)

        @pl.when(nz > 0)
        def _():
            _fix_zero_rows(E, ib, rb)

        cg = cbase + c
        l = cg // cpl
        b0 = (cg % cpl) * CHUNK
        pltpu.async_copy(rb, out_hbm.at[l, pl.ds(b0, CHUNK), pl.ds(0, E)],
                         sw)

    def prefetch(c, first, last, ib, rb, sg, sw):
        @pl.when(jnp.logical_not(jnp.logical_or(first, last)))
        def _():
            wait_write(rb, sw)

        @pl.when(jnp.logical_not(last))
        def _():
            load_and_fire(c, ib, rb, sg)
        return _count_zeros(ib)

    nz0 = prefetch(0, jnp.bool_(True), jnp.bool_(False), idx0, rows0,
                   sg0, sw0)

    def loop_body(i, carry):
        nz0, nz1 = carry
        c0 = 2 * i
        nz1 = prefetch(c0 + 1, i == 0, jnp.bool_(False), idx1, rows1,
                       sg1, sw1)
        process(c0, nz0, idx0, rows0, sg0, sw0)
        nz0 = prefetch(c0 + 2, jnp.bool_(False), i == nch // 2 - 1,
                       idx0, rows0, sg0, sw0)
        process(c0 + 1, nz1, idx1, rows1, sg1, sw1)
        return nz0, nz1

    lax.fori_loop(0, nch // 2, loop_body, (nz0, nz0))

    wait_write(rows0, sw0)
    wait_write(rows1, sw1)


def kernel(input, table, max_steps):
    B, L = input.shape
    V, E = table.shape
    N = B * L
    cpl = B // CHUNK
    nch = N // CHUNK // NW

    idx_flat = input.T.reshape(N).astype(jnp.int32)

    mesh = plsc.VectorSubcoreMesh(core_axis_name="c", subcore_axis_name="s")

    trm = pl.kernel(
        functools.partial(_retile_body, V),
        out_type=jax.ShapeDtypeStruct((V * E,), jnp.float32),
        mesh=mesh,
        compiler_params=pltpu.CompilerParams(use_tc_tiling_on_sc=True,
                                             needs_layout_passes=False),
        scratch_types=[
            pltpu.VMEM((32, GCOLS), jnp.float32),
            pltpu.VMEM((32, GCOLS), jnp.float32),
            pltpu.VMEM((32 * GCOLS,), jnp.float32),
            pltpu.VMEM((32 * GCOLS,), jnp.float32),
            pltpu.VMEM((32, 64), jnp.float32),
            pltpu.VMEM((32 * 64,), jnp.float32),
            pltpu.SemaphoreType.DMA,
            pltpu.SemaphoreType.DMA,
            pltpu.SemaphoreType.DMA,
            pltpu.SemaphoreType.DMA,
            pltpu.SemaphoreType.DMA,
        ],
    )(table.T)
    table_rm = trm.reshape(V, E)

    body = functools.partial(_gather_body, E, cpl, nch)
    out = pl.kernel(
        body,
        out_type=jax.ShapeDtypeStruct((L, B, PADW), jnp.float32),
        mesh=mesh,
        compiler_params=pltpu.CompilerParams(use_tc_tiling_on_sc=False,
                                             needs_layout_passes=False),
        scratch_types=[
            pltpu.VMEM((CHUNK,), jnp.int32),
            pltpu.VMEM((CHUNK,), jnp.int32),
            pltpu.VMEM((CHUNK, E), jnp.float32),
            pltpu.VMEM((CHUNK, E), jnp.float32),
            pltpu.SemaphoreType.DMA,
            pltpu.SemaphoreType.DMA,
            pltpu.SemaphoreType.DMA,
            pltpu.SemaphoreType.DMA,
        ],
    )(idx_flat, table_rm)
    return out[:, :, :E]


# speedup vs baseline: 1.4695x; 1.3359x over previous
import functools

import jax
import jax.numpy as jnp
from jax import lax
from jax.experimental import pallas as pl
from jax.experimental.pallas import tpu as pltpu
from jax.experimental.pallas import tpu_sc as plsc

NC = 2
NS = 16
NW = NC * NS

CHUNK = 512
GATHER = 128
PADW = 128

GCOLS = 512
NG = 61


def _transpose_group(in_v, out_v, ncol):
    lanes = lax.iota(jnp.int32, 16)

    def body(f0, carry):
        fe_v = (f0 + lanes) & 31
        for cb in range(ncol // 16):
            c0 = 16 * cb
            v = plsc.load_gather(in_v, [fe_v, c0 + lanes])
            plsc.store_scatter(out_v, [(c0 * 32 + 32 * lanes) + fe_v], v)
        return carry

    lax.fori_loop(0, 32, body, jnp.int32(0))


def _retile_body(V, tT_hbm, trm_hbm, in0, in1, out0, out1, inp, outp,
                 si0, si1, so0, so1, sp):
    wid = lax.axis_index("s") * NC + lax.axis_index("c")
    vfull = (V // GCOLS) * GCOLS

    def gcol(t):
        return (wid + 32 * t) * GCOLS

    def load(t, iv, si):
        pltpu.async_copy(tT_hbm.at[:, pl.ds(gcol(t), GCOLS)], iv, si)

    def wait_load(iv, si):
        pltpu.make_async_copy(tT_hbm.at[:, pl.ds(0, GCOLS)], iv, si).wait()

    def wait_write(ov, so):
        pltpu.make_async_copy(trm_hbm.at[pl.ds(0, 32 * GCOLS)], ov,
                              so).wait()

    def process(t, iv, ov, si, so):
        wait_load(iv, si)
        _transpose_group(iv, ov, GCOLS)
        pltpu.async_copy(ov, trm_hbm.at[pl.ds(gcol(t) * 32, 32 * GCOLS)],
                         so)

    load(0, in0, si0)

    def loop_body(i, carry):
        t0 = 2 * i
        load(t0 + 1, in1, si1)

        @pl.when(i > 0)
        def _():
            wait_write(out0, so0)
        process(t0, in0, out0, si0, so0)

        @pl.when(t0 + 2 < NG)
        def _():
            load(t0 + 2, in0, si0)

        @pl.when(i > 0)
        def _():
            wait_write(out1, so1)
        process(t0 + 1, in1, out1, si1, so1)
        return carry

    lax.fori_loop(0, NG // 2, loop_body, jnp.int32(0))
    wait_write(out0, so0)
    process(NG - 1, in0, out0, si0, so0)
    wait_write(out0, so0)
    wait_write(out1, so1)

    @pl.when(wid == 0)
    def _():
        pltpu.sync_copy(tT_hbm.at[:, pl.ds(vfull - GCOLS, GCOLS)], in0)
        _transpose_group(in0, out0, GCOLS)
        pltpu.sync_copy(out0,
                        trm_hbm.at[pl.ds((vfull - GCOLS) * 32, 32 * GCOLS)])

        tail = V - vfull
        pltpu.sync_copy(tT_hbm.at[:, pl.ds(vfull, tail)], inp)
        _transpose_group(inp, outp, tail)
        pltpu.sync_copy(outp, trm_hbm.at[pl.ds(vfull * 32, 32 * tail)])


def _count_zeros(idx_v):
    ones = jnp.ones((16,), jnp.int32)
    zer = jnp.zeros((16,), jnp.int32)

    def cnt_body(i, acc):
        v = idx_v[pl.ds(i * 16, 16)]
        return acc + jnp.sum(jnp.where(v == 0, ones, zer))

    return lax.fori_loop(0, CHUNK // 16, cnt_body, jnp.int32(0))


def _fix_zero_rows(E, idx_v, rows_v):
    zeros = jnp.zeros((16,), jnp.float32)

    def fix_body(i, carry):
        v = idx_v[pl.ds(i * 16, 16)]
        m = v == 0
        rowids = lax.iota(jnp.int32, 16) + i * 16
        for col in range(E):
            colids = jnp.full((16,), col, jnp.int32)
            plsc.store_scatter(rows_v, [rowids, colids], zeros, mask=m)
        return carry

    lax.fori_loop(0, CHUNK // 16, fix_body, jnp.int32(0))


def _gather_body(E, cpl, nch, idx_hbm, table_hbm, out_hbm,
                 idx0, idx1, rows0, rows1, sg0, sg1, sw0, sw1):
    wid = lax.axis_index("s") * NC + lax.axis_index("c")
    cbase = wid * nch

    def load_and_fire(c, ib, rb, sg):
        pltpu.sync_copy(idx_hbm.at[pl.ds((cbase + c) * CHUNK, CHUNK)], ib)
        for j in range(CHUNK // GATHER):
            pltpu.async_copy(
                table_hbm.at[ib.at[pl.ds(j * GATHER, GATHER)]],
                rb.at[pl.ds(j * GATHER, GATHER)], sg)

    def wait_gathers(rb, sg):
        pltpu.make_async_copy(table_hbm.at[pl.ds(0, CHUNK)], rb, sg).wait()

    def wait_write(rb, sw):
        pltpu.make_async_copy(table_hbm.at[pl.ds(0, CHUNK)], rb, sw).wait()

    def process(c, nz, ib, rb, sg, sw):
        wait_gathers(rb, sg)

        @pl.when(nz > 0)
        def _():
            _fix_zero_rows(E, ib, rb)

        cg = cbase + c
        l = cg // cpl
        b0 = (cg % cpl) * CHUNK
        pltpu.async_copy(rb, out_hbm.at[l, pl.ds(b0, CHUNK), pl.ds(0, E)],
                         sw)

    def prefetch(c, first, last, ib, rb, sg, sw):
        @pl.when(jnp.logical_not(jnp.logical_or(first, last)))
        def _():
            wait_write(rb, sw)

        @pl.when(jnp.logical_not(last))
        def _():
            load_and_fire(c, ib, rb, sg)
        return _count_zeros(ib)

    nz0 = prefetch(0, jnp.bool_(True), jnp.bool_(False), idx0, rows0,
                   sg0, sw0)

    def loop_body(i, carry):
        nz0, nz1 = carry
        c0 = 2 * i
        nz1 = prefetch(c0 + 1, i == 0, jnp.bool_(False), idx1, rows1,
                       sg1, sw1)
        process(c0, nz0, idx0, rows0, sg0, sw0)
        nz0 = prefetch(c0 + 2, jnp.bool_(False), i == nch // 2 - 1,
                       idx0, rows0, sg0, sw0)
        process(c0 + 1, nz1, idx1, rows1, sg1, sw1)
        return nz0, nz1

    lax.fori_loop(0, nch // 2, loop_body, (nz0, nz0))

    wait_write(rows0, sw0)
    wait_write(rows1, sw1)


def kernel(input, table, max_steps):
    B, L = input.shape
    V, E = table.shape
    N = B * L
    cpl = B // CHUNK
    nch = N // CHUNK // NW

    idx_flat = input.T.reshape(N).astype(jnp.int32)

    mesh = plsc.VectorSubcoreMesh(core_axis_name="c", subcore_axis_name="s")

    trm = pl.kernel(
        functools.partial(_retile_body, V),
        out_type=jax.ShapeDtypeStruct((V * E,), jnp.float32),
        mesh=mesh,
        compiler_params=pltpu.CompilerParams(use_tc_tiling_on_sc=True,
                                             needs_layout_passes=False),
        scratch_types=[
            pltpu.VMEM((32, GCOLS), jnp.float32),
            pltpu.VMEM((32, GCOLS), jnp.float32),
            pltpu.VMEM((32 * GCOLS,), jnp.float32),
            pltpu.VMEM((32 * GCOLS,), jnp.float32),
            pltpu.VMEM((32, 64), jnp.float32),
            pltpu.VMEM((32 * 64,), jnp.float32),
            pltpu.SemaphoreType.DMA,
            pltpu.SemaphoreType.DMA,
            pltpu.SemaphoreType.DMA,
            pltpu.SemaphoreType.DMA,
            pltpu.SemaphoreType.DMA,
        ],
    )(table.T)
    table_rm = trm.reshape(V, E)

    body = functools.partial(_gather_body, E, cpl, nch)
    out = pl.kernel(
        body,
        out_type=jax.ShapeDtypeStruct((L, B, PADW), jnp.float32),
        mesh=mesh,
        compiler_params=pltpu.CompilerParams(use_tc_tiling_on_sc=False,
                                             needs_layout_passes=False),
        scratch_types=[
            pltpu.VMEM((CHUNK,), jnp.int32),
            pltpu.VMEM((CHUNK,), jnp.int32),
            pltpu.VMEM((CHUNK, E), jnp.float32),
            pltpu.VMEM((CHUNK, E), jnp.float32),
            pltpu.SemaphoreType.DMA,
            pltpu.SemaphoreType.DMA,
            pltpu.SemaphoreType.DMA,
            pltpu.SemaphoreType.DMA,
        ],
    )(idx_flat, table_rm)
    return out[:, :, :E]


# speedup vs baseline: 2.1477x; 1.4615x over previous
import functools

import jax
import jax.numpy as jnp
from jax import lax
from jax.experimental import pallas as pl
from jax.experimental.pallas import tpu as pltpu
from jax.experimental.pallas import tpu_sc as plsc

NC = 2
NS = 16
NW = NC * NS

CHUNK = 512
GATHER = 128
PADW = 128

GCOLS = 512
NG = 61


def _transpose_group(in_v, out_v, ncol):
    lanes = lax.iota(jnp.int32, 16)

    @plsc.parallel_loop(0, 32, unroll=2)
    def _(f0):
        fe_v = (f0 + lanes) & 31
        for cb in range(ncol // 16):
            c0 = 16 * cb
            v = plsc.load_gather(in_v, [fe_v, c0 + lanes])
            plsc.store_scatter(out_v, [(c0 * 32 + 32 * lanes) + fe_v], v)


def _retile_body(V, tT_hbm, trm_hbm, in0, in1, out0, out1, inp, outp,
                 si0, si1, so0, so1, sp):
    wid = lax.axis_index("s") * NC + lax.axis_index("c")
    vfull = (V // GCOLS) * GCOLS

    def gcol(t):
        return (wid + 32 * t) * GCOLS

    def load(t, iv, si):
        pltpu.async_copy(tT_hbm.at[:, pl.ds(gcol(t), GCOLS)], iv, si)

    def wait_load(iv, si):
        pltpu.make_async_copy(tT_hbm.at[:, pl.ds(0, GCOLS)], iv, si).wait()

    def wait_write(ov, so):
        pltpu.make_async_copy(trm_hbm.at[pl.ds(0, 32 * GCOLS)], ov,
                              so).wait()

    def process(t, iv, ov, si, so):
        wait_load(iv, si)
        _transpose_group(iv, ov, GCOLS)
        pltpu.async_copy(ov, trm_hbm.at[pl.ds(gcol(t) * 32, 32 * GCOLS)],
                         so)

    load(0, in0, si0)

    def loop_body(i, carry):
        t0 = 2 * i
        load(t0 + 1, in1, si1)

        @pl.when(i > 0)
        def _():
            wait_write(out0, so0)
        process(t0, in0, out0, si0, so0)

        @pl.when(t0 + 2 < NG)
        def _():
            load(t0 + 2, in0, si0)

        @pl.when(i > 0)
        def _():
            wait_write(out1, so1)
        process(t0 + 1, in1, out1, si1, so1)
        return carry

    lax.fori_loop(0, NG // 2, loop_body, jnp.int32(0))
    wait_write(out0, so0)
    process(NG - 1, in0, out0, si0, so0)
    wait_write(out0, so0)
    wait_write(out1, so1)

    @pl.when(wid == 0)
    def _():
        pltpu.sync_copy(tT_hbm.at[:, pl.ds(vfull - GCOLS, GCOLS)], in0)
        _transpose_group(in0, out0, GCOLS)
        pltpu.sync_copy(out0,
                        trm_hbm.at[pl.ds((vfull - GCOLS) * 32, 32 * GCOLS)])

        tail = V - vfull
        pltpu.sync_copy(tT_hbm.at[:, pl.ds(vfull, tail)], inp)
        _transpose_group(inp, outp, tail)
        pltpu.sync_copy(outp, trm_hbm.at[pl.ds(vfull * 32, 32 * tail)])


def _count_zeros(idx_v):
    ones = jnp.ones((16,), jnp.int32)
    zer = jnp.zeros((16,), jnp.int32)

    def cnt_body(i, acc):
        v = idx_v[pl.ds(i * 16, 16)]
        return acc + jnp.sum(jnp.where(v == 0, ones, zer))

    return lax.fori_loop(0, CHUNK // 16, cnt_body, jnp.int32(0))


def _fix_zero_rows(E, idx_v, rows_v):
    zeros = jnp.zeros((16,), jnp.float32)

    def fix_body(i, carry):
        v = idx_v[pl.ds(i * 16, 16)]
        m = v == 0
        rowids = lax.iota(jnp.int32, 16) + i * 16
        for col in range(E):
            colids = jnp.full((16,), col, jnp.int32)
            plsc.store_scatter(rows_v, [rowids, colids], zeros, mask=m)
        return carry

    lax.fori_loop(0, CHUNK // 16, fix_body, jnp.int32(0))


def _gather_body(E, cpl, nch, idx_hbm, table_hbm, out_hbm,
                 idx0, idx1, rows0, rows1, sg0, sg1, sw0, sw1):
    wid = lax.axis_index("s") * NC + lax.axis_index("c")
    cbase = wid * nch

    def load_and_fire(c, ib, rb, sg):
        pltpu.sync_copy(idx_hbm.at[pl.ds((cbase + c) * CHUNK, CHUNK)], ib)
        for j in range(CHUNK // GATHER):
            pltpu.async_copy(
                table_hbm.at[ib.at[pl.ds(j * GATHER, GATHER)]],
                rb.at[pl.ds(j * GATHER, GATHER)], sg)

    def wait_gathers(rb, sg):
        pltpu.make_async_copy(table_hbm.at[pl.ds(0, CHUNK)], rb, sg).wait()

    def wait_write(rb, sw):
        pltpu.make_async_copy(table_hbm.at[pl.ds(0, CHUNK)], rb, sw).wait()

    def process(c, nz, ib, rb, sg, sw):
        wait_gathers(rb, sg)

        @pl.when(nz > 0)
        def _():
            _fix_zero_rows(E, ib, rb)

        cg = cbase + c
        l = cg // cpl
        b0 = (cg % cpl) * CHUNK
        pltpu.async_copy(rb, out_hbm.at[l, pl.ds(b0, CHUNK), pl.ds(0, E)],
                         sw)

    def prefetch(c, first, last, ib, rb, sg, sw):
        @pl.when(jnp.logical_not(jnp.logical_or(first, last)))
        def _():
            wait_write(rb, sw)

        @pl.when(jnp.logical_not(last))
        def _():
            load_and_fire(c, ib, rb, sg)
        return _count_zeros(ib)

    nz0 = prefetch(0, jnp.bool_(True), jnp.bool_(False), idx0, rows0,
                   sg0, sw0)

    def loop_body(i, carry):
        nz0, nz1 = carry
        c0 = 2 * i
        nz1 = prefetch(c0 + 1, i == 0, jnp.bool_(False), idx1, rows1,
                       sg1, sw1)
        process(c0, nz0, idx0, rows0, sg0, sw0)
        nz0 = prefetch(c0 + 2, jnp.bool_(False), i == nch // 2 - 1,
                       idx0, rows0, sg0, sw0)
        process(c0 + 1, nz1, idx1, rows1, sg1, sw1)
        return nz0, nz1

    lax.fori_loop(0, nch // 2, loop_body, (nz0, nz0))

    wait_write(rows0, sw0)
    wait_write(rows1, sw1)


def kernel(input, table, max_steps):
    B, L = input.shape
    V, E = table.shape
    N = B * L
    cpl = B // CHUNK
    nch = N // CHUNK // NW

    idx_flat = input.T.reshape(N).astype(jnp.int32)

    mesh = plsc.VectorSubcoreMesh(core_axis_name="c", subcore_axis_name="s")

    trm = pl.kernel(
        functools.partial(_retile_body, V),
        out_type=jax.ShapeDtypeStruct((V * E,), jnp.float32),
        mesh=mesh,
        compiler_params=pltpu.CompilerParams(use_tc_tiling_on_sc=True,
                                             needs_layout_passes=False),
        scratch_types=[
            pltpu.VMEM((32, GCOLS), jnp.float32),
            pltpu.VMEM((32, GCOLS), jnp.float32),
            pltpu.VMEM((32 * GCOLS,), jnp.float32),
            pltpu.VMEM((32 * GCOLS,), jnp.float32),
            pltpu.VMEM((32, 64), jnp.float32),
            pltpu.VMEM((32 * 64,), jnp.float32),
            pltpu.SemaphoreType.DMA,
            pltpu.SemaphoreType.DMA,
            pltpu.SemaphoreType.DMA,
            pltpu.SemaphoreType.DMA,
            pltpu.SemaphoreType.DMA,
        ],
    )(table.T)
    table_rm = trm.reshape(V, E)

    body = functools.partial(_gather_body, E, cpl, nch)
    out = pl.kernel(
        body,
        out_type=jax.ShapeDtypeStruct((L, B, PADW), jnp.float32),
        mesh=mesh,
        compiler_params=pltpu.CompilerParams(use_tc_tiling_on_sc=False,
                                             needs_layout_passes=False),
        scratch_types=[
            pltpu.VMEM((CHUNK,), jnp.int32),
            pltpu.VMEM((CHUNK,), jnp.int32),
            pltpu.VMEM((CHUNK, E), jnp.float32),
            pltpu.VMEM((CHUNK, E), jnp.float32),
            pltpu.SemaphoreType.DMA,
            pltpu.SemaphoreType.DMA,
            pltpu.SemaphoreType.DMA,
            pltpu.SemaphoreType.DMA,
        ],
    )(idx_flat, table_rm)
    return out[:, :, :E]


# speedup vs baseline: 2.1791x; 1.0146x over previous
import functools

import jax
import jax.numpy as jnp
from jax import lax
from jax.experimental import pallas as pl
from jax.experimental.pallas import tpu as pltpu
from jax.experimental.pallas import tpu_sc as plsc

NC = 2
NS = 16
NW = NC * NS

CHUNK = 640
GATHER = 128
PADW = 128

GCOLS = 512
NG = 61


def _transpose_group(in_v, out_v, ncol):
    lanes = lax.iota(jnp.int32, 16)

    @plsc.parallel_loop(0, 32, unroll=2)
    def _(f0):
        fe_v = (f0 + lanes) & 31
        for cb in range(ncol // 16):
            c0 = 16 * cb
            v = plsc.load_gather(in_v, [fe_v, c0 + lanes])
            plsc.store_scatter(out_v, [(c0 * 32 + 32 * lanes) + fe_v], v)


def _retile_body(V, tT_hbm, trm_hbm, in0, in1, out0, out1, inp, outp,
                 si0, si1, so0, so1, sp):
    wid = lax.axis_index("s") * NC + lax.axis_index("c")
    vfull = (V // GCOLS) * GCOLS

    def gcol(t):
        return (wid + 32 * t) * GCOLS

    def load(t, iv, si):
        pltpu.async_copy(tT_hbm.at[:, pl.ds(gcol(t), GCOLS)], iv, si)

    def wait_load(iv, si):
        pltpu.make_async_copy(tT_hbm.at[:, pl.ds(0, GCOLS)], iv, si).wait()

    def wait_write(ov, so):
        pltpu.make_async_copy(trm_hbm.at[pl.ds(0, 32 * GCOLS)], ov,
                              so).wait()

    def process(t, iv, ov, si, so):
        wait_load(iv, si)
        _transpose_group(iv, ov, GCOLS)
        pltpu.async_copy(ov, trm_hbm.at[pl.ds(gcol(t) * 32, 32 * GCOLS)],
                         so)

    load(0, in0, si0)

    def loop_body(i, carry):
        t0 = 2 * i
        load(t0 + 1, in1, si1)

        @pl.when(i > 0)
        def _():
            wait_write(out0, so0)
        process(t0, in0, out0, si0, so0)

        @pl.when(t0 + 2 < NG)
        def _():
            load(t0 + 2, in0, si0)

        @pl.when(i > 0)
        def _():
            wait_write(out1, so1)
        process(t0 + 1, in1, out1, si1, so1)
        return carry

    lax.fori_loop(0, NG // 2, loop_body, jnp.int32(0))
    wait_write(out0, so0)
    process(NG - 1, in0, out0, si0, so0)
    wait_write(out0, so0)
    wait_write(out1, so1)

    @pl.when(wid == 0)
    def _():
        pltpu.sync_copy(tT_hbm.at[:, pl.ds(vfull - GCOLS, GCOLS)], in0)
        _transpose_group(in0, out0, GCOLS)
        pltpu.sync_copy(out0,
                        trm_hbm.at[pl.ds((vfull - GCOLS) * 32, 32 * GCOLS)])

        tail = V - vfull
        pltpu.sync_copy(tT_hbm.at[:, pl.ds(vfull, tail)], inp)
        _transpose_group(inp, outp, tail)
        pltpu.sync_copy(outp, trm_hbm.at[pl.ds(vfull * 32, 32 * tail)])


def _count_zeros(idx_v):
    ones = jnp.ones((16,), jnp.int32)
    zer = jnp.zeros((16,), jnp.int32)

    def cnt_body(i, acc):
        v = idx_v[pl.ds(i * 16, 16)]
        return acc + jnp.sum(jnp.where(v == 0, ones, zer))

    return lax.fori_loop(0, CHUNK // 16, cnt_body, jnp.int32(0))


def _fix_zero_rows(E, idx_v, rows_v):
    zeros = jnp.zeros((16,), jnp.float32)

    def fix_body(i, carry):
        v = idx_v[pl.ds(i * 16, 16)]
        m = v == 0
        rowids = lax.iota(jnp.int32, 16) + i * 16
        for col in range(E):
            colids = jnp.full((16,), col, jnp.int32)
            plsc.store_scatter(rows_v, [rowids, colids], zeros, mask=m)
        return carry

    lax.fori_loop(0, CHUNK // 16, fix_body, jnp.int32(0))


def _gather_body(E, cpl, nch, idx_hbm, table_hbm, out_hbm,
                 idx0, idx1, rows0, rows1, sg0, sg1, sw0, sw1):
    wid = lax.axis_index("s") * NC + lax.axis_index("c")
    cbase = wid * nch

    def load_and_fire(c, ib, rb, sg):
        pltpu.sync_copy(idx_hbm.at[pl.ds((cbase + c) * CHUNK, CHUNK)], ib)
        for j in range(CHUNK // GATHER):
            pltpu.async_copy(
                table_hbm.at[ib.at[pl.ds(j * GATHER, GATHER)]],
                rb.at[pl.ds(j * GATHER, GATHER)], sg)

    def wait_gathers(rb, sg):
        pltpu.make_async_copy(table_hbm.at[pl.ds(0, CHUNK)], rb, sg).wait()

    def wait_write(rb, sw):
        pltpu.make_async_copy(table_hbm.at[pl.ds(0, CHUNK)], rb, sw).wait()

    def process(c, nz, ib, rb, sg, sw):
        wait_gathers(rb, sg)

        @pl.when(nz > 0)
        def _():
            _fix_zero_rows(E, ib, rb)

        cg = cbase + c
        l = cg // cpl
        b0 = (cg % cpl) * CHUNK
        pltpu.async_copy(rb, out_hbm.at[l, pl.ds(b0, CHUNK), pl.ds(0, E)],
                         sw)

    def prefetch(c, first, last, ib, rb, sg, sw):
        @pl.when(jnp.logical_not(jnp.logical_or(first, last)))
        def _():
            wait_write(rb, sw)

        @pl.when(jnp.logical_not(last))
        def _():
            load_and_fire(c, ib, rb, sg)
        return _count_zeros(ib)

    nz0 = prefetch(0, jnp.bool_(True), jnp.bool_(False), idx0, rows0,
                   sg0, sw0)

    def loop_body(i, carry):
        nz0, nz1 = carry
        c0 = 2 * i
        nz1 = prefetch(c0 + 1, i == 0, jnp.bool_(False), idx1, rows1,
                       sg1, sw1)
        process(c0, nz0, idx0, rows0, sg0, sw0)
        nz0 = prefetch(c0 + 2, jnp.bool_(False), i == nch // 2 - 1,
                       idx0, rows0, sg0, sw0)
        process(c0 + 1, nz1, idx1, rows1, sg1, sw1)
        return nz0, nz1

    lax.fori_loop(0, nch // 2, loop_body, (nz0, nz0))

    wait_write(rows0, sw0)
    wait_write(rows1, sw1)


def kernel(input, table, max_steps):
    B, L = input.shape
    V, E = table.shape
    N = B * L
    cpl = B // CHUNK
    nch = N // CHUNK // NW

    idx_flat = input.T.reshape(N).astype(jnp.int32)

    mesh = plsc.VectorSubcoreMesh(core_axis_name="c", subcore_axis_name="s")

    trm = pl.kernel(
        functools.partial(_retile_body, V),
        out_type=jax.ShapeDtypeStruct((V * E,), jnp.float32),
        mesh=mesh,
        compiler_params=pltpu.CompilerParams(use_tc_tiling_on_sc=True,
                                             needs_layout_passes=False),
        scratch_types=[
            pltpu.VMEM((32, GCOLS), jnp.float32),
            pltpu.VMEM((32, GCOLS), jnp.float32),
            pltpu.VMEM((32 * GCOLS,), jnp.float32),
            pltpu.VMEM((32 * GCOLS,), jnp.float32),
            pltpu.VMEM((32, 64), jnp.float32),
            pltpu.VMEM((32 * 64,), jnp.float32),
            pltpu.SemaphoreType.DMA,
            pltpu.SemaphoreType.DMA,
            pltpu.SemaphoreType.DMA,
            pltpu.SemaphoreType.DMA,
            pltpu.SemaphoreType.DMA,
        ],
    )(table.T)
    table_rm = trm.reshape(V, E)

    body = functools.partial(_gather_body, E, cpl, nch)
    out = pl.kernel(
        body,
        out_type=jax.ShapeDtypeStruct((L, B, PADW), jnp.float32),
        mesh=mesh,
        compiler_params=pltpu.CompilerParams(use_tc_tiling_on_sc=False,
                                             needs_layout_passes=False),
        scratch_types=[
            pltpu.VMEM((CHUNK,), jnp.int32),
            pltpu.VMEM((CHUNK,), jnp.int32),
            pltpu.VMEM((CHUNK, E), jnp.float32),
            pltpu.VMEM((CHUNK, E), jnp.float32),
            pltpu.SemaphoreType.DMA,
            pltpu.SemaphoreType.DMA,
            pltpu.SemaphoreType.DMA,
            pltpu.SemaphoreType.DMA,
        ],
    )(idx_flat, table_rm)
    return out[:, :, :E]
